# baseline (speedup 1.0000x reference)
import functools

import jax
import jax.numpy as jnp
from jax import lax
from jax.experimental import pallas as pl
from jax.experimental.pallas import tpu as pltpu
from jax.experimental.pallas import tpu_sc as plsc

BATCH = 16384
NUM_NUMERIC = 13
NUM_CATEGORICAL = 26
STATE = 64
CARD = 100000
OUT_CH = (NUM_NUMERIC + NUM_CATEGORICAL) * STATE
NUM_COLS = NUM_NUMERIC * STATE

NC = 2
NS = 16
NW = NC * NS
B_PER_W = BATCH // NW
G_CHUNK = 128
N_SUB = B_PER_W // G_CHUNK
N_UNITS = NUM_CATEGORICAL * N_SUB
N_PAIRS = N_UNITS // 2


def _body(xn_t, xc_t, w_hbm, bias_hbm, emb_p, out_hbm,
          idx_v, idx_g, gbuf0, gbuf1, tbuf0, tbuf1, nbuf, xcv, w_v, b_v,
          sem_g0, sem_g1, sem_o0, sem_o1, sem_n):
    wid = lax.axis_index("s") * NC + lax.axis_index("c")
    base = wid * B_PER_W

    pltpu.sync_copy(xc_t.at[:, pl.ds(base, B_PER_W)], idx_v)
    pltpu.sync_copy(xn_t.at[:, pl.ds(base, B_PER_W)], xcv)
    pltpu.sync_copy(w_hbm, w_v)
    pltpu.sync_copy(bias_hbm, b_v)

    def _shift(n, _):
        t = n // NB16
        k = n % NB16
        idx_g[t, pl.ds(k * 16, 16)] = jax.lax.shift_right_logical(
            idx_v[t, pl.ds(k * 16, 16)], 1)
        return 0
    NB16 = B_PER_W // 16
    lax.fori_loop(0, NUM_CATEGORICAL * NB16, _shift, 0)

    def _g_copy(u, gbuf, sem):
        t = u // N_SUB
        s = u % N_SUB
        return pltpu.make_async_copy(
            emb_p.at[t].at[idx_g.at[t, pl.ds(s * G_CHUNK, G_CHUNK)]],
            gbuf, sem)

    def _o_copy(u, tbuf, sem):
        t = u // N_SUB
        s = u % N_SUB
        return pltpu.make_async_copy(
            tbuf, out_hbm.at[pl.ds(NUM_COLS + t * STATE, STATE),
                             pl.ds(base + s * G_CHUNK, G_CHUNK)], sem)

    def _transpose(u, gbuf, tbuf):
        t = u // N_SUB
        s = u % N_SUB

        def _k(k, _):
            rows = jax.lax.broadcasted_iota(jnp.int32, (16,), 0) + k * 16
            par = (idx_v[t, pl.ds(s * G_CHUNK + k * 16, 16)] &
                   jnp.full((16,), 1, jnp.int32)) * STATE

            def _c4(c4, _, rows=rows, par=par):
                for cc in range(16):
                    c = c4 * 16 + cc
                    tbuf[c, pl.ds(k * 16, 16)] = plsc.load_gather(
                        gbuf, [rows, par + c])
                return 0
            lax.fori_loop(0, STATE // 16, _c4, 0)
            return 0
        lax.fori_loop(0, G_CHUNK // 16, _k, 0)

    def _numeric(m):
        i = m // N_SUB
        sn = m % N_SUB

        def _k(k, _):
            x16 = xcv[i, pl.ds(sn * G_CHUNK + k * 16, 16)]

            def _c4(c4, _, x16=x16):
                ii = jnp.full((16,), i, jnp.int32)
                for cc in range(16):
                    c = c4 * 16 + cc
                    cv = jnp.full((16,), c, jnp.int32)
                    ws = plsc.load_gather(w_v, [ii, cv])
                    bs = plsc.load_gather(b_v, [ii, cv])
                    nbuf[c, pl.ds(k * 16, 16)] = x16 * ws + bs
                return 0
            lax.fori_loop(0, STATE // 16, _c4, 0)
            return 0
        lax.fori_loop(0, G_CHUNK // 16, _k, 0)
        return pltpu.make_async_copy(
            nbuf, out_hbm.at[pl.ds(i * STATE, STATE),
                             pl.ds(base + sn * G_CHUNK, G_CHUNK)], sem_n)

    _g_copy(0, gbuf0, sem_g0).start()
    _g_copy(1, gbuf1, sem_g1).start()

    def _pair(m, _):
        uA = 2 * m
        uB = 2 * m + 1

        _g_copy(uA, gbuf0, sem_g0).wait()

        @pl.when(m > 0)
        def _():
            _o_copy(uA - 2, tbuf0, sem_o0).wait()
        _transpose(uA, gbuf0, tbuf0)

        @pl.when(uA + 2 < N_UNITS)
        def _():
            _g_copy(uA + 2, gbuf0, sem_g0).start()
        _o_copy(uA, tbuf0, sem_o0).start()

        _g_copy(uB, gbuf1, sem_g1).wait()

        @pl.when(m > 0)
        def _():
            _o_copy(uB - 2, tbuf1, sem_o1).wait()
        _transpose(uB, gbuf1, tbuf1)

        @pl.when(uB + 2 < N_UNITS)
        def _():
            _g_copy(uB + 2, gbuf1, sem_g1).start()
        _o_copy(uB, tbuf1, sem_o1).start()

        @pl.when(m > 0)
        def _():
            _numeric(m - 1).wait()
        _numeric(m).start()
        return 0

    lax.fori_loop(0, N_PAIRS, _pair, 0)

    _o_copy(N_UNITS - 2, tbuf0, sem_o0).wait()
    _o_copy(N_UNITS - 1, tbuf1, sem_o1).wait()
    _numeric(N_PAIRS - 1).wait()


@jax.jit
def _run(xn_t, xc_t, W_num, b_num, emb_p):
    mesh = plsc.VectorSubcoreMesh(core_axis_name="c", subcore_axis_name="s")
    return pl.kernel(
        _body,
        mesh=mesh,
        compiler_params=pltpu.CompilerParams(needs_layout_passes=False),
        out_type=jax.ShapeDtypeStruct((OUT_CH, BATCH), jnp.float32),
        scratch_types=[
            pltpu.VMEM((NUM_CATEGORICAL, B_PER_W), jnp.int32),
            pltpu.VMEM((NUM_CATEGORICAL, B_PER_W), jnp.int32),
            pltpu.VMEM((G_CHUNK, 2 * STATE), jnp.float32),
            pltpu.VMEM((G_CHUNK, 2 * STATE), jnp.float32),
            pltpu.VMEM((STATE, G_CHUNK), jnp.float32),
            pltpu.VMEM((STATE, G_CHUNK), jnp.float32),
            pltpu.VMEM((STATE, G_CHUNK), jnp.float32),
            pltpu.VMEM((NUM_NUMERIC, B_PER_W), jnp.float32),
            pltpu.VMEM((NUM_NUMERIC, STATE), jnp.float32),
            pltpu.VMEM((NUM_NUMERIC, STATE), jnp.float32),
            pltpu.SemaphoreType.DMA,
            pltpu.SemaphoreType.DMA,
            pltpu.SemaphoreType.DMA,
            pltpu.SemaphoreType.DMA,
            pltpu.SemaphoreType.DMA,
        ],
    )(xn_t, xc_t, W_num, b_num, emb_p)


def kernel(x_numeric, x_categorical, W_num, b_num, emb_tables):
    xn_t = x_numeric.T
    xc_t = x_categorical.astype(jnp.int32).T
    emb_p = emb_tables.reshape(NUM_CATEGORICAL, CARD // 2, 2 * STATE)
    out_t = _run(xn_t, xc_t, W_num, b_num, emb_p)
    return out_t.T


# speedup vs baseline: 1.3043x; 1.3043x over previous
import functools

import jax
import jax.numpy as jnp
from jax import lax
from jax.experimental import pallas as pl
from jax.experimental.pallas import tpu as pltpu
from jax.experimental.pallas import tpu_sc as plsc

BATCH = 16384
NUM_NUMERIC = 13
NUM_CATEGORICAL = 26
STATE = 64
CARD = 100000
OUT_CH = (NUM_NUMERIC + NUM_CATEGORICAL) * STATE
NUM_COLS = NUM_NUMERIC * STATE

NC = 2
NS = 16
NW = NC * NS
B_PER_W = BATCH // NW
G_CHUNK = 128
N_SUB = B_PER_W // G_CHUNK
N_UNITS = NUM_CATEGORICAL * N_SUB
N_PAIRS = N_UNITS // 2


def _body(xn_t, xc_t, w_hbm, bias_hbm, emb_p, out_hbm,
          idx_v, idx_g, gbuf0, gbuf1, tbuf0, tbuf1, nbuf, xcv, w_v, b_v,
          sem_g0, sem_g1, sem_o0, sem_o1, sem_n):
    wid = lax.axis_index("s") * NC + lax.axis_index("c")
    base = wid * B_PER_W

    pltpu.sync_copy(xc_t.at[:, pl.ds(base, B_PER_W)], idx_v)
    pltpu.sync_copy(xn_t.at[:, pl.ds(base, B_PER_W)], xcv)
    pltpu.sync_copy(w_hbm, w_v)
    pltpu.sync_copy(bias_hbm, b_v)

    def _shift(n, _):
        t = n // NB16
        k = n % NB16
        idx_g[t, pl.ds(k * 16, 16)] = jax.lax.shift_right_logical(
            idx_v[t, pl.ds(k * 16, 16)], 1)
        return 0
    NB16 = B_PER_W // 16
    lax.fori_loop(0, NUM_CATEGORICAL * NB16, _shift, 0)

    def _g_copy(u, gbuf, sem):
        t = u // N_SUB
        s = u % N_SUB
        return pltpu.make_async_copy(
            emb_p.at[t].at[idx_g.at[t, pl.ds(s * G_CHUNK, G_CHUNK)]],
            gbuf, sem)

    def _o_copy(u, tbuf, sem):
        t = u // N_SUB
        s = u % N_SUB
        return pltpu.make_async_copy(
            tbuf, out_hbm.at[pl.ds(NUM_COLS + t * STATE, STATE),
                             pl.ds(base + s * G_CHUNK, G_CHUNK)], sem)

    def _transpose(u, gbuf, tbuf):
        t = u // N_SUB
        s = u % N_SUB

        @plsc.parallel_loop(0, G_CHUNK // 16)
        def _k(k):
            rows = jax.lax.broadcasted_iota(jnp.int32, (16,), 0) + k * 16
            par = (idx_v[t, pl.ds(s * G_CHUNK + k * 16, 16)] &
                   jnp.full((16,), 1, jnp.int32)) * STATE
            for c4 in range(STATE // 16):
                cols = [par + (c4 * 16 + j) for j in range(16)]
                vals = [plsc.load_gather(gbuf, [rows, cv]) for cv in cols]
                for j in range(16):
                    tbuf[c4 * 16 + j, pl.ds(k * 16, 16)] = vals[j]

    def _numeric(m):
        i = m // N_SUB
        sn = m % N_SUB
        ii = jnp.full((16,), i, jnp.int32)
        for c4 in range(STATE // 16):
            cvs = [jnp.full((16,), c4 * 16 + j, jnp.int32) for j in range(16)]
            wss = [plsc.load_gather(w_v, [ii, cv]) for cv in cvs]
            bss = [plsc.load_gather(b_v, [ii, cv]) for cv in cvs]

            @plsc.parallel_loop(0, G_CHUNK // 16)
            def _k(k, c4=c4, wss=wss, bss=bss):
                x16 = xcv[i, pl.ds(sn * G_CHUNK + k * 16, 16)]
                for j in range(16):
                    nbuf[c4 * 16 + j, pl.ds(k * 16, 16)] = (
                        x16 * wss[j] + bss[j])
        return pltpu.make_async_copy(
            nbuf, out_hbm.at[pl.ds(i * STATE, STATE),
                             pl.ds(base + sn * G_CHUNK, G_CHUNK)], sem_n)

    _g_copy(0, gbuf0, sem_g0).start()
    _g_copy(1, gbuf1, sem_g1).start()

    def _pair(m, _):
        uA = 2 * m
        uB = 2 * m + 1

        _g_copy(uA, gbuf0, sem_g0).wait()

        @pl.when(m > 0)
        def _():
            _o_copy(uA - 2, tbuf0, sem_o0).wait()
        _transpose(uA, gbuf0, tbuf0)

        @pl.when(uA + 2 < N_UNITS)
        def _():
            _g_copy(uA + 2, gbuf0, sem_g0).start()
        _o_copy(uA, tbuf0, sem_o0).start()

        _g_copy(uB, gbuf1, sem_g1).wait()

        @pl.when(m > 0)
        def _():
            _o_copy(uB - 2, tbuf1, sem_o1).wait()
        _transpose(uB, gbuf1, tbuf1)

        @pl.when(uB + 2 < N_UNITS)
        def _():
            _g_copy(uB + 2, gbuf1, sem_g1).start()
        _o_copy(uB, tbuf1, sem_o1).start()

        @pl.when(m > 0)
        def _():
            _numeric(m - 1).wait()
        _numeric(m).start()
        return 0

    lax.fori_loop(0, N_PAIRS, _pair, 0)

    _o_copy(N_UNITS - 2, tbuf0, sem_o0).wait()
    _o_copy(N_UNITS - 1, tbuf1, sem_o1).wait()
    _numeric(N_PAIRS - 1).wait()


@jax.jit
def _run(xn_t, xc_t, W_num, b_num, emb_p):
    mesh = plsc.VectorSubcoreMesh(core_axis_name="c", subcore_axis_name="s")
    return pl.kernel(
        _body,
        mesh=mesh,
        compiler_params=pltpu.CompilerParams(needs_layout_passes=False),
        out_type=jax.ShapeDtypeStruct((OUT_CH, BATCH), jnp.float32),
        scratch_types=[
            pltpu.VMEM((NUM_CATEGORICAL, B_PER_W), jnp.int32),
            pltpu.VMEM((NUM_CATEGORICAL, B_PER_W), jnp.int32),
            pltpu.VMEM((G_CHUNK, 2 * STATE), jnp.float32),
            pltpu.VMEM((G_CHUNK, 2 * STATE), jnp.float32),
            pltpu.VMEM((STATE, G_CHUNK), jnp.float32),
            pltpu.VMEM((STATE, G_CHUNK), jnp.float32),
            pltpu.VMEM((STATE, G_CHUNK), jnp.float32),
            pltpu.VMEM((NUM_NUMERIC, B_PER_W), jnp.float32),
            pltpu.VMEM((NUM_NUMERIC, STATE), jnp.float32),
            pltpu.VMEM((NUM_NUMERIC, STATE), jnp.float32),
            pltpu.SemaphoreType.DMA,
            pltpu.SemaphoreType.DMA,
            pltpu.SemaphoreType.DMA,
            pltpu.SemaphoreType.DMA,
            pltpu.SemaphoreType.DMA,
        ],
    )(xn_t, xc_t, W_num, b_num, emb_p)


def kernel(x_numeric, x_categorical, W_num, b_num, emb_tables):
    xn_t = x_numeric.T
    xc_t = x_categorical.astype(jnp.int32).T
    emb_p = emb_tables.reshape(NUM_CATEGORICAL, CARD // 2, 2 * STATE)
    out_t = _run(xn_t, xc_t, W_num, b_num, emb_p)
    return out_t.T
